# baseline (speedup 1.0000x reference)
import functools

import jax
import jax.numpy as jnp
from jax import lax
from jax.experimental import pallas as pl
from jax.experimental.pallas import tpu as pltpu
from jax.experimental.pallas import tpu_sc as plsc

_NUM_CORES = 2
_NUM_SUBCORES = 16
_NUM_WORKERS = _NUM_CORES * _NUM_SUBCORES


@functools.lru_cache(maxsize=None)
def _broadcast_kernel(batch, seq, hidden):
    rows_per_worker = seq // _NUM_WORKERS
    mesh = plsc.VectorSubcoreMesh(core_axis_name="c", subcore_axis_name="s")

    @functools.partial(
        pl.kernel,
        mesh=mesh,
        out_type=jax.ShapeDtypeStruct((batch, seq, hidden), jnp.float32),
        scratch_types=[pltpu.SemaphoreType.DMA],
    )
    def k(table_hbm, out_hbm, sem):
        wid = lax.axis_index("s") * _NUM_CORES + lax.axis_index("c")
        base = wid * rows_per_worker
        src = table_hbm.at[pl.ds(base, rows_per_worker), :]
        handles = [
            pltpu.async_copy(
                src, out_hbm.at[b, pl.ds(base, rows_per_worker), :], sem)
            for b in range(batch)
        ]
        for h in handles:
            h.wait()

    return k


def kernel(x, pos_table):
    batch = x.shape[0]
    seq, hidden = pos_table.shape
    return _broadcast_kernel(batch, seq, hidden)(pos_table)


# speedup vs baseline: 55.1639x; 55.1639x over previous
import functools

import jax
import jax.numpy as jnp
from jax import lax
from jax.experimental import pallas as pl
from jax.experimental.pallas import tpu as pltpu
from jax.experimental.pallas import tpu_sc as plsc

_NUM_CORES = 2
_NUM_SUBCORES = 16
_NUM_WORKERS = _NUM_CORES * _NUM_SUBCORES
_BUF_ROWS = 56


@functools.lru_cache(maxsize=None)
def _broadcast_kernel(batch, seq, hidden):
    rows_per_worker = seq // _NUM_WORKERS
    chunk_rows = []
    r = rows_per_worker
    while r > 0:
        c = min(r, _BUF_ROWS)
        chunk_rows.append(c)
        r -= c
    chunk_offs = [sum(chunk_rows[:i]) for i in range(len(chunk_rows))]
    num_chunks = len(chunk_rows)
    mesh = plsc.VectorSubcoreMesh(core_axis_name="c", subcore_axis_name="s")

    @functools.partial(
        pl.kernel,
        mesh=mesh,
        out_type=jax.ShapeDtypeStruct((batch, seq, hidden), jnp.float32),
        scratch_types=[
            pltpu.VMEM((_BUF_ROWS, hidden), jnp.float32),
            pltpu.VMEM((_BUF_ROWS, hidden), jnp.float32),
            pltpu.SemaphoreType.DMA,
            pltpu.SemaphoreType.DMA,
            pltpu.SemaphoreType.DMA,
        ],
    )
    def k(table_hbm, out_hbm, buf0, buf1, ld, st0, st1):
        wid = lax.axis_index("s") * _NUM_CORES + lax.axis_index("c")
        base = wid * rows_per_worker
        bufs = (buf0, buf1)
        sts = (st0, st1)
        loads = [None] * num_chunks
        stores = [None] * num_chunks

        def start_load(i):
            n = chunk_rows[i]
            return pltpu.async_copy(
                table_hbm.at[pl.ds(base + chunk_offs[i], n), :],
                bufs[i % 2].at[pl.ds(0, n), :], ld)

        loads[0] = start_load(0)
        for i in range(num_chunks):
            n = chunk_rows[i]
            loads[i].wait()
            if i + 1 < num_chunks:
                if i >= 1:
                    for h in stores[i - 1]:
                        h.wait()
                loads[i + 1] = start_load(i + 1)
            row0 = base + chunk_offs[i]
            stores[i] = [
                pltpu.async_copy(
                    bufs[i % 2].at[pl.ds(0, n), :],
                    out_hbm.at[b, pl.ds(row0, n), :], sts[i % 2])
                for b in range(batch)
            ]
        for i in (num_chunks - 2, num_chunks - 1):
            if i >= 0:
                for h in stores[i]:
                    h.wait()

    return k


def kernel(x, pos_table):
    batch = x.shape[0]
    seq, hidden = pos_table.shape
    return _broadcast_kernel(batch, seq, hidden)(pos_table)
